# baseline (speedup 1.0000x reference)
import functools
import math

import jax
import jax.numpy as jnp
from jax import lax
from jax.experimental import pallas as pl
from jax.experimental.pallas import tpu as pltpu
from jax.experimental.pallas import tpu_sc as plsc

_NC = 2
_NS = 16
_NW = _NC * _NS
_CHUNK = 128
_ZROWS = 64


def _pick_wave(k, c=32):
    wmax = 10 if c <= 16 else 8
    for w in range(wmax, 0, -1):
        if k % w == 0:
            return w
    return 1


@functools.lru_cache(maxsize=None)
def _make_spmm(ns_pad, nd_pad, c, k):
    wave = _pick_wave(k, c)
    nwaves = k // wave
    rpt = nd_pad // _NS
    spt = ns_pad // _NS
    assert rpt % _ZROWS == 0

    def body(tbl, src, dst, out, idx_s, idx_d, rows, zbuf, tspm, acc, gsem, ssem):
        cid = lax.axis_index("c")
        sid = lax.axis_index("s")
        wid = sid * _NC + cid

        h_is = pltpu.async_copy(src.at[wid], idx_s, gsem)
        h_id = pltpu.async_copy(dst.at[wid], idx_d, gsem)
        h_tb = pltpu.async_copy(tbl.at[pl.ds(sid * spt, spt)],
                                tspm.at[pl.ds(sid * spt, spt)], gsem)

        zero = jnp.zeros((16,), jnp.float32)

        def zrow(i, carry):
            for bb in range(c // 16):
                zbuf[i, pl.ds(bb * 16, 16)] = zero
            return carry

        lax.fori_loop(0, _ZROWS, zrow, 0)
        h_is.wait()
        h_id.wait()

        zh = []
        for i in range(rpt // _ZROWS):
            zh.append(pltpu.async_copy(
                zbuf, acc.at[pl.ds(sid * rpt + i * _ZROWS, _ZROWS)], ssem))
        for h in zh:
            h.wait()
        h_tb.wait()
        plsc.subcore_barrier()

        def g_desc(w, b, grp):
            return pltpu.make_async_copy(
                tspm.at[idx_s.at[w * wave + b]], rows.at[grp, b], gsem)

        def s_desc(w, b, grp):
            return pltpu.make_async_copy(
                rows.at[grp, b], acc.at[idx_d.at[w * wave + b]], ssem)

        for b in range(wave):
            g_desc(0, b, 0).start()

        def loop_body(w, carry):
            grp = lax.rem(w, 2)
            ngrp = lax.rem(w + 1, 2)

            @pl.when(w >= 1)
            def _():
                for b in range(wave):
                    s_desc(w - 1, b, ngrp).wait()

            @pl.when(w + 1 < nwaves)
            def _():
                for b in range(wave):
                    g_desc(w + 1, b, ngrp).start()

            for b in range(wave):
                g_desc(w, b, grp).wait()
            for b in range(wave):
                s_desc(w, b, grp).start(add=True)
            return carry

        lax.fori_loop(0, nwaves, loop_body, 0)
        lgrp = (nwaves - 1) % 2
        for b in range(wave):
            s_desc(nwaves - 1, b, lgrp).wait()
        plsc.subcore_barrier()

        pltpu.sync_copy(acc.at[pl.ds(sid * rpt, rpt)],
                        out.at[cid].at[pl.ds(sid * rpt, rpt)])

    return pl.kernel(
        body,
        out_type=jax.ShapeDtypeStruct((2, nd_pad, c), jnp.float32),
        mesh=plsc.VectorSubcoreMesh(core_axis_name="c", subcore_axis_name="s"),
        scratch_types=[
            pltpu.VMEM((k, _CHUNK), jnp.int32),
            pltpu.VMEM((k, _CHUNK), jnp.int32),
            pltpu.VMEM((2, wave, _CHUNK, c), jnp.float32),
            pltpu.VMEM((_ZROWS, c), jnp.float32),
            pltpu.VMEM_SHARED((ns_pad, c), jnp.float32),
            pltpu.VMEM_SHARED((nd_pad, c), jnp.float32),
            pltpu.SemaphoreType.DMA,
            pltpu.SemaphoreType.DMA,
        ],
        compiler_params=pltpu.CompilerParams(use_tc_tiling_on_sc=False),
    )


def _tc(fn, out_shapes, *args):
    n = len(args)

    def body(*refs):
        vals = fn(*[r[...] for r in refs[:n]])
        if not isinstance(vals, (tuple, list)):
            vals = (vals,)
        for r, v in zip(refs[n:], vals):
            r[...] = v

    res = pl.pallas_call(
        body,
        out_shape=[jax.ShapeDtypeStruct(s, jnp.float32) for s in out_shapes],
    )(*args)
    return res if len(out_shapes) > 1 else res[0]


def _dot(a, b):
    return jnp.dot(a, b, preferred_element_type=jnp.float32)


def _sig(z):
    return 1.0 / (1.0 + jnp.exp(-z))


def _relu(z):
    return jax.nn.relu(z)


def _prep_edges(src, dst, n_true, nd_pad, k):
    epad = _NW * k * _CHUNK
    e = src.shape[0]
    assert epad >= e and nd_pad > n_true
    src_p = jnp.zeros((epad,), jnp.int32).at[:e].set(src.astype(jnp.int32))
    dst_p = jnp.full((epad,), n_true, jnp.int32).at[:e].set(dst.astype(jnp.int32))
    return src_p.reshape(_NW, k, _CHUNK), dst_p.reshape(_NW, k, _CHUNK)


def _k_of(e):
    k = max(1, math.ceil(e / (_NW * _CHUNK)))
    while k > 8 and _pick_wave(k) < 3:
        k += 1
    return k


def _make_p(srcdst, ns_pad, nd_pad, k):
    s3, d3 = srcdst

    def P(tbl):
        c = tbl.shape[1]
        return _make_spmm(ns_pad, nd_pad, c, k)(tbl, s3, d3)

    return P


def _dfa_pre(P, z, p, m1, pp1=None):
    n = z.shape[0]
    c1 = p["conv1"]
    cin1 = pp1 is not None
    if cin1:
        pp = pp1
    else:
        if m1 is None:
            m1 = _tc(lambda z_, w: _dot(z_, w), [(n, 32)], z, c1["Wn"])
        pp = P(m1)
    x1 = h = t = qq = a = out1 = None
    for i in range(3):
        q = p["irn%d" % i]
        w10, b10 = q["conv1_0"]["W"], q["conv1_0"]["b"].reshape(1, -1)
        wn00, wn11 = q["conv0_0"]["Wn"], q["conv1_1"]["Wn"]
        if i == 0:
            def ta0(pp_, z_, wn1_, ws1, b1, w10_, b10_, wn00_, wn11_):
                s = pp_[0] + pp_[1]
                conv1_agg = _dot(s[:, :1], wn1_) if cin1 else s
                x1_ = _relu(conv1_agg + _dot(z_, ws1) + b1)
                t_ = _relu(_dot(x1_, w10_) + b10_)
                mc = jnp.concatenate([_dot(x1_, wn00_), _dot(t_, wn11_)], axis=1)
                return x1_, t_, mc

            x1, t, mcat = _tc(ta0, [(n, 32), (n, 16), (n, 32)], pp, z,
                              c1["Wn"], c1["Ws"], c1["b"].reshape(1, -1),
                              w10, b10, wn00, wn11)
            h = x1
        else:
            qp = p["irn%d" % (i - 1)]
            ws01, b01 = qp["conv0_1"]["Ws"], qp["conv0_1"]["b"].reshape(1, -1)

            def ta(qq_, a_, o1_, h_, ws01_, b01_, w10_, b10_, wn00_, wn11_):
                out0 = qq_[0] + qq_[1] + _dot(a_, ws01_) + b01_
                hn = jnp.concatenate([out0, o1_], axis=1) + h_
                t_ = _relu(_dot(hn, w10_) + b10_)
                mc = jnp.concatenate([_dot(hn, wn00_), _dot(t_, wn11_)], axis=1)
                return hn, t_, mc

            h, t, mcat = _tc(ta, [(n, 32), (n, 16), (n, 32)], qq, a, out1, h,
                             ws01, b01, w10, b10, wn00, wn11)
        pc = P(mcat)
        ws00, b00 = q["conv0_0"]["Ws"], q["conv0_0"]["b"].reshape(1, -1)
        ws11, b11 = q["conv1_1"]["Ws"], q["conv1_1"]["b"].reshape(1, -1)
        w12, b12 = q["conv1_2"]["W"], q["conv1_2"]["b"].reshape(1, -1)
        wn01 = q["conv0_1"]["Wn"]

        def tb(pc_, h_, t_, ws00_, b00_, ws11_, b11_, w12_, b12_, wn01_):
            s = pc_[0] + pc_[1]
            a_ = _relu(s[:, :16] + _dot(h_, ws00_) + b00_)
            t2 = _relu(s[:, 16:] + _dot(t_, ws11_) + b11_)
            o1 = _dot(t2, w12_) + b12_
            return a_, o1, _dot(a_, wn01_)

        a, out1, m01 = _tc(tb, [(n, 16)] * 3, pc, h, t,
                           ws00, b00, ws11, b11, w12, b12, wn01)
        qq = P(m01)
    q2 = p["irn2"]
    ws01, b01 = q2["conv0_1"]["Ws"], q2["conv0_1"]["b"].reshape(1, -1)
    c2 = p["conv2"]

    def tcf(qq_, a_, o1_, h_, x1_, ws01_, b01_, wn2_):
        out0 = qq_[0] + qq_[1] + _dot(a_, ws01_) + b01_
        hx = jnp.concatenate([out0, o1_], axis=1) + h_ + x1_
        return hx, _dot(hx, wn2_)

    hx, m2 = _tc(tcf, [(n, 32), (n, 32)], qq, a, out1, h, x1,
                 ws01, b01, c2["Wn"])
    return hx, P(m2), c2


def kernel(x, params, noise, edge_index0, edge_index1, edge_index2,
           down01, down12, node_ids1, pov_ids):
    n0, n1, n2 = x.shape[0], node_ids1.shape[0], noise.shape[0]
    p0, p1p, p2p = 10240, 3072, 1024

    k0, k1, k2 = _k_of(edge_index0.shape[1]), _k_of(edge_index1.shape[1]), \
        _k_of(edge_index2.shape[1])
    kd01, kd12, kup = _k_of(n0), _k_of(n1), _k_of(n1)
    ed0 = _prep_edges(edge_index0[0], edge_index0[1], n0, p0, k0)
    ed1 = _prep_edges(edge_index1[0], edge_index1[1], n1, p1p, k1)
    ed2 = _prep_edges(edge_index2[0], edge_index2[1], n2, p2p, k2)
    ar0 = jnp.arange(n0, dtype=jnp.int32)
    ar1 = jnp.arange(n1, dtype=jnp.int32)
    P0 = _make_p(ed0, p0, p0, k0)
    P1 = _make_p(ed1, p1p, p1p, k1)
    P2 = _make_p(ed2, p2p, p2p, k2)
    Pd01 = _make_p(_prep_edges(ar0, down01, n1, p1p, kd01), p0, p1p, kd01)
    Pd12 = _make_p(_prep_edges(ar1, down12, n2, p2p, kd12), p1p, p2p, kd12)
    Pup = _make_p(_prep_edges(down12, ar1, n1, p1p, kup), p2p, p1p, kup)

    xp = jnp.zeros((p0, x.shape[1]), jnp.float32).at[:n0].set(x)
    x16 = jnp.zeros((p0, 16), jnp.float32).at[:n0, :1].set(x)
    noisep = jnp.full((p2p, noise.shape[1]), 0.5, jnp.float32).at[:n2].set(noise)

    hx, pc, c2 = _dfa_pre(P0, xp, params["enc_dfa0"], None, pp1=P0(x16))

    def down_a(pc_, hx_, ws2, b2, w):
        return _dot(pc_[0] + pc_[1] + _dot(hx_, ws2) + b2, w)

    hw = _tc(down_a, [(p0, 32)], pc, hx, c2["Ws"], c2["b"].reshape(1, -1),
             params["down01"]["W"])
    pd = Pd01(hw)

    def down_b(pd_, db, wn1):
        h_ = pd_[0] + pd_[1] + db
        return h_, _dot(h_, wn1)

    h1, m1 = _tc(down_b, [(p1p, 32)] * 2, pd,
                 params["down01"]["b"].reshape(1, -1),
                 params["enc_dfa1"]["conv1"]["Wn"])
    hx, pc, c2 = _dfa_pre(P1, h1, params["enc_dfa1"], m1)
    hw = _tc(down_a, [(p1p, 32)], pc, hx, c2["Ws"], c2["b"].reshape(1, -1),
             params["down12"]["W"])
    pd = Pd12(hw)
    h2, m2 = _tc(down_b, [(p2p, 32)] * 2, pd,
                 params["down12"]["b"].reshape(1, -1),
                 params["enc_dfa2"]["conv1"]["Wn"])
    hx, pc, c2 = _dfa_pre(P2, h2, params["enc_dfa2"], m2)

    def likf(pc_, hx_, ws2, b2, nz, mu, logs):
        y = pc_[0] + pc_[1] + _dot(hx_, ws2) + b2
        yh = y + (nz - 0.5)
        s = jnp.exp(logs)
        l = _sig((yh + 0.5 - mu) / s) - _sig((yh - 0.5 - mu) / s)
        return jnp.clip(l, 1e-9, 1.0)[:n2], yh

    lik, y_hat = _tc(likf, [(n2, 32), (p2p, 32)], pc, hx,
                     c2["Ws"], c2["b"].reshape(1, -1), noisep,
                     params["eb"]["mu"].reshape(1, -1),
                     params["eb"]["logs"].reshape(1, -1))

    g = Pup(y_hat)

    def upf(g_, w, b, wn1):
        u0 = _dot(g_[0] + g_[1], w) + b
        return u0, _dot(u0, wn1)

    u0, m1d = _tc(upf, [(p1p, 32)] * 2, g, params["up"]["W"],
                  params["up"]["b"].reshape(1, -1),
                  params["dec_dfa"]["conv1"]["Wn"])
    hx, pc, c2 = _dfa_pre(P1, u0, params["dec_dfa"], m1d)

    dc = params["dec_conv"]

    def dcf(pc_, hx_, ws2, b2, wn):
        u_ = pc_[0] + pc_[1] + _dot(hx_, ws2) + b2
        return u_, _dot(u_, wn)

    u_, mdc = _tc(dcf, [(p1p, 32)] * 2, pc, hx, c2["Ws"],
                  c2["b"].reshape(1, -1), dc["Wn"])
    pdc = P1(mdc)

    cl = params["cls"]

    def clsf(pdc_, u2_, wsdc, bdc, wncls):
        u1 = pdc_[0] + pdc_[1] + _dot(u2_, wsdc) + bdc
        m = jnp.concatenate(
            [_dot(u1, wncls), jnp.zeros((u1.shape[0], 15), jnp.float32)], axis=1)
        return u1, m

    u1, mcls = _tc(clsf, [(p1p, 32), (p1p, 16)], pdc, u_,
                   dc["Ws"], dc["b"].reshape(1, -1), cl["Wn"])
    pcls = P1(mcls)

    def finf(pcls_, u1_, wscls, bcls, ids, pov):
        xc = (pcls_[0] + pcls_[1])[:, :1] + _dot(u1_, wscls) + bcls
        msk = jnp.any(ids == pov, axis=1, keepdims=True)
        return u1_[:n1] * msk.astype(jnp.float32), xc[:n1]

    u_out, x_cls = _tc(finf, [(n1, 32), (n1, 1)], pcls, u1,
                       cl["Ws"], cl["b"].reshape(1, -1),
                       node_ids1.astype(jnp.int32).reshape(-1, 1),
                       pov_ids.astype(jnp.int32).reshape(1, -1))

    return (u_out, lik, x_cls)


# speedup vs baseline: 1.0248x; 1.0248x over previous
import functools
import math

import jax
import jax.numpy as jnp
from jax import lax
from jax.experimental import pallas as pl
from jax.experimental.pallas import tpu as pltpu
from jax.experimental.pallas import tpu_sc as plsc

_NC = 2
_NS = 16
_NW = _NC * _NS
_CHUNK = 128
_ZROWS = 64


def _pick_wave(k, c=32):
    for w in range(8, 0, -1):
        if k % w == 0:
            return w
    return 1


@functools.lru_cache(maxsize=None)
def _make_spmm(ns_pad, nd_pad, c, k):
    wave = _pick_wave(k, c)
    nwaves = k // wave
    rpt = nd_pad // _NS
    spt = ns_pad // _NS
    assert rpt % _ZROWS == 0

    def body(tbl, src, dst, out, idx_s, idx_d, rows, zbuf, tspm, acc, gsem, ssem):
        cid = lax.axis_index("c")
        sid = lax.axis_index("s")
        wid = sid * _NC + cid

        h_is = pltpu.async_copy(src.at[wid], idx_s, gsem)
        h_id = pltpu.async_copy(dst.at[wid], idx_d, gsem)
        h_tb = pltpu.async_copy(tbl.at[pl.ds(sid * spt, spt)],
                                tspm.at[pl.ds(sid * spt, spt)], gsem)

        zero = jnp.zeros((16,), jnp.float32)

        def zrow(i, carry):
            for bb in range(c // 16):
                zbuf[i, pl.ds(bb * 16, 16)] = zero
            return carry

        lax.fori_loop(0, _ZROWS, zrow, 0)
        h_is.wait()
        h_id.wait()

        zh = []
        for i in range(rpt // _ZROWS):
            zh.append(pltpu.async_copy(
                zbuf, acc.at[pl.ds(sid * rpt + i * _ZROWS, _ZROWS)], ssem))
        for h in zh:
            h.wait()
        h_tb.wait()
        plsc.subcore_barrier()

        def g_desc(w, b, grp):
            return pltpu.make_async_copy(
                tspm.at[idx_s.at[w * wave + b]], rows.at[grp, b], gsem)

        def s_desc(w, b, grp):
            return pltpu.make_async_copy(
                rows.at[grp, b], acc.at[idx_d.at[w * wave + b]], ssem)

        for b in range(wave):
            g_desc(0, b, 0).start()

        def loop_body(w, carry):
            grp = lax.rem(w, 2)
            ngrp = lax.rem(w + 1, 2)

            @pl.when(w >= 1)
            def _():
                for b in range(wave):
                    s_desc(w - 1, b, ngrp).wait()

            @pl.when(w + 1 < nwaves)
            def _():
                for b in range(wave):
                    g_desc(w + 1, b, ngrp).start()

            for b in range(wave):
                g_desc(w, b, grp).wait()
                s_desc(w, b, grp).start(add=True)
            return carry

        lax.fori_loop(0, nwaves, loop_body, 0)
        lgrp = (nwaves - 1) % 2
        for b in range(wave):
            s_desc(nwaves - 1, b, lgrp).wait()
        plsc.subcore_barrier()

        pltpu.sync_copy(acc.at[pl.ds(sid * rpt, rpt)],
                        out.at[cid].at[pl.ds(sid * rpt, rpt)])

    return pl.kernel(
        body,
        out_type=jax.ShapeDtypeStruct((2, nd_pad, c), jnp.float32),
        mesh=plsc.VectorSubcoreMesh(core_axis_name="c", subcore_axis_name="s"),
        scratch_types=[
            pltpu.VMEM((k, _CHUNK), jnp.int32),
            pltpu.VMEM((k, _CHUNK), jnp.int32),
            pltpu.VMEM((2, wave, _CHUNK, c), jnp.float32),
            pltpu.VMEM((_ZROWS, c), jnp.float32),
            pltpu.VMEM_SHARED((ns_pad, c), jnp.float32),
            pltpu.VMEM_SHARED((nd_pad, c), jnp.float32),
            pltpu.SemaphoreType.DMA,
            pltpu.SemaphoreType.DMA,
        ],
        compiler_params=pltpu.CompilerParams(use_tc_tiling_on_sc=False),
    )


def _tc(fn, out_shapes, *args):
    n = len(args)

    def body(*refs):
        vals = fn(*[r[...] for r in refs[:n]])
        if not isinstance(vals, (tuple, list)):
            vals = (vals,)
        for r, v in zip(refs[n:], vals):
            r[...] = v

    res = pl.pallas_call(
        body,
        out_shape=[jax.ShapeDtypeStruct(s, jnp.float32) for s in out_shapes],
    )(*args)
    return res if len(out_shapes) > 1 else res[0]


def _dot(a, b):
    return jnp.dot(a, b, preferred_element_type=jnp.float32)


def _sig(z):
    return 1.0 / (1.0 + jnp.exp(-z))


def _relu(z):
    return jax.nn.relu(z)


def _prep_edges(src, dst, n_true, nd_pad, k):
    epad = _NW * k * _CHUNK
    e = src.shape[0]
    assert epad >= e and nd_pad > n_true
    src_p = jnp.zeros((epad,), jnp.int32).at[:e].set(src.astype(jnp.int32))
    dst_p = jnp.full((epad,), n_true, jnp.int32).at[:e].set(dst.astype(jnp.int32))
    return src_p.reshape(_NW, k, _CHUNK), dst_p.reshape(_NW, k, _CHUNK)


def _k_of(e):
    k = max(1, math.ceil(e / (_NW * _CHUNK)))
    while k > 8 and _pick_wave(k) < 3:
        k += 1
    return k


def _make_p(srcdst, ns_pad, nd_pad, k):
    s3, d3 = srcdst

    def P(tbl):
        c = tbl.shape[1]
        return _make_spmm(ns_pad, nd_pad, c, k)(tbl, s3, d3)

    return P


def _dfa_pre(P, z, p, m1, pp1=None):
    n = z.shape[0]
    c1 = p["conv1"]
    cin1 = pp1 is not None
    if cin1:
        pp = pp1
    else:
        if m1 is None:
            m1 = _tc(lambda z_, w: _dot(z_, w), [(n, 32)], z, c1["Wn"])
        pp = P(m1)
    x1 = h = t = qq = a = out1 = None
    for i in range(3):
        q = p["irn%d" % i]
        w10, b10 = q["conv1_0"]["W"], q["conv1_0"]["b"].reshape(1, -1)
        wn00, wn11 = q["conv0_0"]["Wn"], q["conv1_1"]["Wn"]
        if i == 0:
            def ta0(pp_, z_, wn1_, ws1, b1, w10_, b10_, wn00_, wn11_):
                s = pp_[0] + pp_[1]
                conv1_agg = _dot(s[:, :1], wn1_) if cin1 else s
                x1_ = _relu(conv1_agg + _dot(z_, ws1) + b1)
                t_ = _relu(_dot(x1_, w10_) + b10_)
                mc = jnp.concatenate([_dot(x1_, wn00_), _dot(t_, wn11_)], axis=1)
                return x1_, t_, mc

            x1, t, mcat = _tc(ta0, [(n, 32), (n, 16), (n, 32)], pp, z,
                              c1["Wn"], c1["Ws"], c1["b"].reshape(1, -1),
                              w10, b10, wn00, wn11)
            h = x1
        else:
            qp = p["irn%d" % (i - 1)]
            ws01, b01 = qp["conv0_1"]["Ws"], qp["conv0_1"]["b"].reshape(1, -1)

            def ta(qq_, a_, o1_, h_, ws01_, b01_, w10_, b10_, wn00_, wn11_):
                out0 = qq_[0] + qq_[1] + _dot(a_, ws01_) + b01_
                hn = jnp.concatenate([out0, o1_], axis=1) + h_
                t_ = _relu(_dot(hn, w10_) + b10_)
                mc = jnp.concatenate([_dot(hn, wn00_), _dot(t_, wn11_)], axis=1)
                return hn, t_, mc

            h, t, mcat = _tc(ta, [(n, 32), (n, 16), (n, 32)], qq, a, out1, h,
                             ws01, b01, w10, b10, wn00, wn11)
        pc = P(mcat)
        ws00, b00 = q["conv0_0"]["Ws"], q["conv0_0"]["b"].reshape(1, -1)
        ws11, b11 = q["conv1_1"]["Ws"], q["conv1_1"]["b"].reshape(1, -1)
        w12, b12 = q["conv1_2"]["W"], q["conv1_2"]["b"].reshape(1, -1)
        wn01 = q["conv0_1"]["Wn"]

        def tb(pc_, h_, t_, ws00_, b00_, ws11_, b11_, w12_, b12_, wn01_):
            s = pc_[0] + pc_[1]
            a_ = _relu(s[:, :16] + _dot(h_, ws00_) + b00_)
            t2 = _relu(s[:, 16:] + _dot(t_, ws11_) + b11_)
            o1 = _dot(t2, w12_) + b12_
            return a_, o1, _dot(a_, wn01_)

        a, out1, m01 = _tc(tb, [(n, 16)] * 3, pc, h, t,
                           ws00, b00, ws11, b11, w12, b12, wn01)
        qq = P(m01)
    q2 = p["irn2"]
    ws01, b01 = q2["conv0_1"]["Ws"], q2["conv0_1"]["b"].reshape(1, -1)
    c2 = p["conv2"]

    def tcf(qq_, a_, o1_, h_, x1_, ws01_, b01_, wn2_):
        out0 = qq_[0] + qq_[1] + _dot(a_, ws01_) + b01_
        hx = jnp.concatenate([out0, o1_], axis=1) + h_ + x1_
        return hx, _dot(hx, wn2_)

    hx, m2 = _tc(tcf, [(n, 32), (n, 32)], qq, a, out1, h, x1,
                 ws01, b01, c2["Wn"])
    return hx, P(m2), c2


def kernel(x, params, noise, edge_index0, edge_index1, edge_index2,
           down01, down12, node_ids1, pov_ids):
    n0, n1, n2 = x.shape[0], node_ids1.shape[0], noise.shape[0]
    p0, p1p, p2p = 10240, 3072, 1024

    k0, k1, k2 = _k_of(edge_index0.shape[1]), _k_of(edge_index1.shape[1]), \
        _k_of(edge_index2.shape[1])
    kd01, kd12, kup = _k_of(n0), _k_of(n1), _k_of(n1)
    ed0 = _prep_edges(edge_index0[0], edge_index0[1], n0, p0, k0)
    ed1 = _prep_edges(edge_index1[0], edge_index1[1], n1, p1p, k1)
    ed2 = _prep_edges(edge_index2[0], edge_index2[1], n2, p2p, k2)
    ar0 = jnp.arange(n0, dtype=jnp.int32)
    ar1 = jnp.arange(n1, dtype=jnp.int32)
    P0 = _make_p(ed0, p0, p0, k0)
    P1 = _make_p(ed1, p1p, p1p, k1)
    P2 = _make_p(ed2, p2p, p2p, k2)
    Pd01 = _make_p(_prep_edges(ar0, down01, n1, p1p, kd01), p0, p1p, kd01)
    Pd12 = _make_p(_prep_edges(ar1, down12, n2, p2p, kd12), p1p, p2p, kd12)
    Pup = _make_p(_prep_edges(down12, ar1, n1, p1p, kup), p2p, p1p, kup)

    xp = jnp.zeros((p0, x.shape[1]), jnp.float32).at[:n0].set(x)
    x16 = jnp.zeros((p0, 16), jnp.float32).at[:n0, :1].set(x)
    noisep = jnp.full((p2p, noise.shape[1]), 0.5, jnp.float32).at[:n2].set(noise)

    hx, pc, c2 = _dfa_pre(P0, xp, params["enc_dfa0"], None, pp1=P0(x16))

    def down_a(pc_, hx_, ws2, b2, w):
        return _dot(pc_[0] + pc_[1] + _dot(hx_, ws2) + b2, w)

    hw = _tc(down_a, [(p0, 32)], pc, hx, c2["Ws"], c2["b"].reshape(1, -1),
             params["down01"]["W"])
    pd = Pd01(hw)

    def down_b(pd_, db, wn1):
        h_ = pd_[0] + pd_[1] + db
        return h_, _dot(h_, wn1)

    h1, m1 = _tc(down_b, [(p1p, 32)] * 2, pd,
                 params["down01"]["b"].reshape(1, -1),
                 params["enc_dfa1"]["conv1"]["Wn"])
    hx, pc, c2 = _dfa_pre(P1, h1, params["enc_dfa1"], m1)
    hw = _tc(down_a, [(p1p, 32)], pc, hx, c2["Ws"], c2["b"].reshape(1, -1),
             params["down12"]["W"])
    pd = Pd12(hw)
    h2, m2 = _tc(down_b, [(p2p, 32)] * 2, pd,
                 params["down12"]["b"].reshape(1, -1),
                 params["enc_dfa2"]["conv1"]["Wn"])
    hx, pc, c2 = _dfa_pre(P2, h2, params["enc_dfa2"], m2)

    def likf(pc_, hx_, ws2, b2, nz, mu, logs):
        y = pc_[0] + pc_[1] + _dot(hx_, ws2) + b2
        yh = y + (nz - 0.5)
        s = jnp.exp(logs)
        l = _sig((yh + 0.5 - mu) / s) - _sig((yh - 0.5 - mu) / s)
        return jnp.clip(l, 1e-9, 1.0)[:n2], yh

    lik, y_hat = _tc(likf, [(n2, 32), (p2p, 32)], pc, hx,
                     c2["Ws"], c2["b"].reshape(1, -1), noisep,
                     params["eb"]["mu"].reshape(1, -1),
                     params["eb"]["logs"].reshape(1, -1))

    g = Pup(y_hat)

    def upf(g_, w, b, wn1):
        u0 = _dot(g_[0] + g_[1], w) + b
        return u0, _dot(u0, wn1)

    u0, m1d = _tc(upf, [(p1p, 32)] * 2, g, params["up"]["W"],
                  params["up"]["b"].reshape(1, -1),
                  params["dec_dfa"]["conv1"]["Wn"])
    hx, pc, c2 = _dfa_pre(P1, u0, params["dec_dfa"], m1d)

    dc = params["dec_conv"]

    def dcf(pc_, hx_, ws2, b2, wn):
        u_ = pc_[0] + pc_[1] + _dot(hx_, ws2) + b2
        return u_, _dot(u_, wn)

    u_, mdc = _tc(dcf, [(p1p, 32)] * 2, pc, hx, c2["Ws"],
                  c2["b"].reshape(1, -1), dc["Wn"])
    pdc = P1(mdc)

    cl = params["cls"]

    def clsf(pdc_, u2_, wsdc, bdc, wncls):
        u1 = pdc_[0] + pdc_[1] + _dot(u2_, wsdc) + bdc
        m = jnp.concatenate(
            [_dot(u1, wncls), jnp.zeros((u1.shape[0], 15), jnp.float32)], axis=1)
        return u1, m

    u1, mcls = _tc(clsf, [(p1p, 32), (p1p, 16)], pdc, u_,
                   dc["Ws"], dc["b"].reshape(1, -1), cl["Wn"])
    pcls = P1(mcls)

    def finf(pcls_, u1_, wscls, bcls, ids, pov):
        xc = (pcls_[0] + pcls_[1])[:, :1] + _dot(u1_, wscls) + bcls
        msk = jnp.any(ids == pov, axis=1, keepdims=True)
        return u1_[:n1] * msk.astype(jnp.float32), xc[:n1]

    u_out, x_cls = _tc(finf, [(n1, 32), (n1, 1)], pcls, u1,
                       cl["Ws"], cl["b"].reshape(1, -1),
                       node_ids1.astype(jnp.int32).reshape(-1, 1),
                       pov_ids.astype(jnp.int32).reshape(1, -1))

    return (u_out, lik, x_cls)
